# baseline (speedup 1.0000x reference)
import dataclasses
import functools

import jax
import jax.numpy as jnp
from jax import lax
from jax.experimental import pallas as pl
from jax.experimental.pallas import tpu as pltpu
from jax.experimental.pallas import tpu_sc as plsc

P = 32
RES = 128
TEXN = 4 * RES * RES
IMG_H = IMG_W = 256
N = IMG_H * IMG_W
STEP = 2.0 / 255.0
ALMOST_ONE = 0.99999994


def _bitonic_pairs(n):
    pairs = []
    k = 2
    while k <= n:
        j = k // 2
        while j >= 1:
            for i in range(n):
                l = i ^ j
                if l > i:
                    pairs.append((i, l, (i & k) == 0))
            j //= 2
        k *= 2
    return pairs


_PAIRS = _bitonic_pairs(P)


_TILE_H = 8
_GRID_A = IMG_H // _TILE_H


def _b16(x):
    i = lax.bitcast_convert_type(x, jnp.int32)
    lsb = lax.bitwise_and(lax.shift_right_logical(i, 16), 1)
    r = lax.bitwise_and(i + (32767 + lsb), jnp.int32(-65536))
    return lax.bitcast_convert_type(r, jnp.float32)


def _two_sum(a, b):
    s = a + b
    bp = s - a
    ap = s - bp
    return s, (a - ap) + (b - bp)


def _dot3(p0, p1, p2):
    s1, e1 = _two_sum(p0, p1)
    s2, e2 = _two_sum(s1, p2)
    return s2 + (e1 + e2)


def _geom_body(par_ref, o_ref, rb_ref,
               depth_ref, mask_ref, idx_ref, wu_ref, wv_ref):
    gi = pl.program_id(0)
    o = [o_ref[j] for j in range(3)]
    yi = (gi * _TILE_H
          + lax.broadcasted_iota(jnp.int32, (_TILE_H, IMG_W), 0))
    xi = lax.broadcasted_iota(jnp.int32, (_TILE_H, IMG_W), 1)
    sx = xi.astype(jnp.float32) / 255.0
    sy = yi.astype(jnp.float32) / 255.0
    gx = sx - (1.0 - sx)
    gy = sy - (1.0 - sy)
    gxb = _b16(gx)
    gyb = _b16(gy)
    d = [_dot3(gxb * rb_ref[0, k], gyb * rb_ref[1, k],
               jnp.full((_TILE_H, IMG_W), rb_ref[2, k], jnp.float32))
         for k in range(3)]
    db = [_b16(x) for x in d]
    for p in range(P):
        nb = [par_ref[p, j] for j in range(3)]
        nump = par_ref[p, 3]
        B0 = [par_ref[p, 4 + j] for j in range(3)]
        B1 = [par_ref[p, 7 + j] for j in range(3)]
        c = [par_ref[p, 10 + j] for j in range(3)]
        hw0 = par_ref[p, 13]
        hw1 = par_ref[p, 14]
        den = _dot3(nb[0] * db[0], nb[1] * db[1], nb[2] * db[2])
        den = jnp.where(jnp.abs(den) < 1e-8, jnp.float32(1e-8), den)
        dep = nump / den
        wm = [_b16((o[j] + dep * d[j]) - c[j]) for j in range(3)]
        loc0 = _dot3(B0[0] * wm[0], B0[1] * wm[1], B0[2] * wm[2])
        loc1 = _dot3(B1[0] * wm[0], B1[1] * wm[1], B1[2] * wm[2])
        gxp = loc0 / hw0
        gyp = loc1 / hw1
        ip = (jnp.abs(gxp) <= 1.0) & (jnp.abs(gyp) <= 1.0)
        u = (gxp + 1.0) * 0.5 * (RES - 1)
        v = (gyp + 1.0) * 0.5 * (RES - 1)
        u0 = jnp.clip(jnp.floor(u), 0.0, RES - 1)
        v0 = jnp.clip(jnp.floor(v), 0.0, RES - 1)
        depth_ref[p] = dep
        mask_ref[p] = jnp.where(ip, jnp.float32(1.0), jnp.float32(0.0))
        idx_ref[p] = v0.astype(jnp.int32) * RES + u0.astype(jnp.int32)
        wu_ref[p] = jnp.clip(u - u0, 0.0, 1.0)
        wv_ref[p] = jnp.clip(v - v0, 0.0, 1.0)


def _run_geometry(plane_basis, plane_center, plane_wh, cam_R, cam_T):
    origin = -(cam_R.T @ cam_T)
    normals = plane_basis[:, 2, :]
    num = jnp.sum(normals * (plane_center - origin[None]), axis=-1)
    params = jnp.concatenate([
        _b16(normals),
        num[:, None],
        _b16(plane_basis[:, 0, :]),
        _b16(plane_basis[:, 1, :]),
        plane_center,
        plane_wh * 0.5,
    ], axis=1)
    smem = pl.BlockSpec(memory_space=pltpu.SMEM)
    obs = pl.BlockSpec((P, _TILE_H, IMG_W), lambda i: (0, i, 0))
    f32 = jnp.float32
    outs = jax.ShapeDtypeStruct((P, IMG_H, IMG_W), f32)
    return pl.pallas_call(
        _geom_body,
        grid=(_GRID_A,),
        in_specs=[smem] * 3,
        out_specs=[obs] * 5,
        out_shape=[outs, outs,
                   jax.ShapeDtypeStruct((P, IMG_H, IMG_W), jnp.int32),
                   outs, outs],
    )(params, origin, _b16(cam_R))


_NC = 2
_NS = 16
_CHUNK = 4096
_NCHUNK = N // _CHUNK


def _sc_gather_fn():
    mesh = plsc.VectorSubcoreMesh(core_axis_name="c", subcore_axis_name="s")
    f32 = jnp.float32
    out = jax.ShapeDtypeStruct((P, _NCHUNK, 4, _CHUNK), f32)
    cp = pltpu.CompilerParams()
    if "needs_layout_passes" in pltpu.CompilerParams.__dataclass_fields__:
        cp = dataclasses.replace(cp, needs_layout_passes=False)

    @functools.partial(
        pl.kernel, mesh=mesh,
        out_type=out,
        compiler_params=cp,
        scratch_types=[
            pltpu.VMEM((TEXN,), f32),
            pltpu.VMEM((4, _CHUNK), f32),
            pltpu.VMEM((4, _CHUNK), f32),
        ])
    def sc_gather(tex_hbm, in_hbm, out_hbm, tex_v, in_v, out_v):
        w = lax.axis_index("s") * _NC + lax.axis_index("c")
        pltpu.sync_copy(tex_hbm.at[w], tex_v)

        @pl.loop(0, _NCHUNK)
        def _chunk(ci):
            pltpu.sync_copy(in_hbm.at[w, ci], in_v)

            @pl.loop(0, _CHUNK, step=16)
            def _grp(g):
                sl = pl.ds(g, 16)
                i00 = in_v[0, sl].astype(jnp.int32)
                wu = in_v[1, sl]
                wv = in_v[2, sl]
                m = in_v[3, sl]
                u0 = lax.bitwise_and(i00, 127)
                du = jnp.where(u0 < 127, 1, 0).astype(jnp.int32)
                dv = jnp.where(i00 < (RES - 1) * RES, RES, 0).astype(jnp.int32)
                i10 = i00 + du
                i01 = i00 + dv
                i11 = i10 + dv
                w11 = wu * wv
                w10 = wu - w11
                w01 = wv - w11
                w00 = (1.0 - wu) - w01
                for ch in range(4):
                    off = ch * (RES * RES)
                    c00 = plsc.load_gather(tex_v, [i00 + off])
                    c10 = plsc.load_gather(tex_v, [i10 + off])
                    c01 = plsc.load_gather(tex_v, [i01 + off])
                    c11 = plsc.load_gather(tex_v, [i11 + off])
                    val = c00 * w00 + c10 * w10 + c01 * w01 + c11 * w11
                    out_v[ch, sl] = m / (1.0 + jnp.exp(-val))

            pltpu.sync_copy(out_v, out_hbm.at[w, ci])

    return sc_gather


_TILE_CH = 8
_TILE_CW = 128


def _composite_body(d_ref, a_ref, r_ref, g_ref, b_ref,
                    w_ref, si_ref, ri_ref, gi_ref, bi_ref, di_ref):
    d = [d_ref[p] for p in range(P)]
    pay = [jnp.float32(p) + jnp.minimum(a_ref[p], jnp.float32(ALMOST_ONE))
           for p in range(P)]
    for i, l, asc in _PAIRS:
        c = (d[i] < d[l]) | ((d[i] == d[l]) & (pay[i] < pay[l]))
        if not asc:
            c = jnp.logical_not(c)
        d[i], d[l] = jnp.where(c, d[i], d[l]), jnp.where(c, d[l], d[i])
        pay[i], pay[l] = (jnp.where(c, pay[i], pay[l]),
                          jnp.where(c, pay[l], pay[i]))
    t = jnp.ones((_TILE_CH, _TILE_CW), jnp.float32)
    dimg = jnp.zeros((_TILE_CH, _TILE_CW), jnp.float32)
    key2 = []
    for s in range(P):
        idxf = jnp.floor(pay[s])
        alpha = pay[s] - idxf
        wgt = alpha * t
        t = t * (1.0 - alpha)
        w_ref[s] = wgt
        si_ref[s] = pay[s].astype(jnp.int32)
        dimg = dimg + d[s] * wgt
        key2.append(idxf + jnp.minimum(wgt, jnp.float32(ALMOST_ONE)))
    for i, l, asc in _PAIRS:
        lo = jnp.minimum(key2[i], key2[l])
        hi = jnp.maximum(key2[i], key2[l])
        key2[i], key2[l] = (lo, hi) if asc else (hi, lo)
    rimg = jnp.zeros((_TILE_CH, _TILE_CW), jnp.float32)
    gimg = jnp.zeros((_TILE_CH, _TILE_CW), jnp.float32)
    bimg = jnp.zeros((_TILE_CH, _TILE_CW), jnp.float32)
    for p in range(P):
        wo = key2[p] - jnp.float32(p)
        rimg = rimg + r_ref[p] * wo
        gimg = gimg + g_ref[p] * wo
        bimg = bimg + b_ref[p] * wo
    ri_ref[...] = rimg
    gi_ref[...] = gimg
    bi_ref[...] = bimg
    di_ref[...] = dimg


def _run_composite(depth, alpha, red, grn, blu):
    ibs = pl.BlockSpec((P, _TILE_CH, _TILE_CW), lambda i, j: (0, i, j))
    img = pl.BlockSpec((_TILE_CH, _TILE_CW), lambda i, j: (i, j))
    f32 = jnp.float32
    return pl.pallas_call(
        _composite_body,
        grid=(IMG_H // _TILE_CH, IMG_W // _TILE_CW),
        in_specs=[ibs] * 5,
        out_specs=[ibs, ibs, img, img, img, img],
        out_shape=[
            jax.ShapeDtypeStruct((P, IMG_H, IMG_W), f32),
            jax.ShapeDtypeStruct((P, IMG_H, IMG_W), jnp.int32),
            jax.ShapeDtypeStruct((IMG_H, IMG_W), f32),
            jax.ShapeDtypeStruct((IMG_H, IMG_W), f32),
            jax.ShapeDtypeStruct((IMG_H, IMG_W), f32),
            jax.ShapeDtypeStruct((IMG_H, IMG_W), f32),
        ],
    )(depth, alpha, red, grn, blu)


def kernel(plane_content, plane_basis, plane_center, plane_wh, cam_R, cam_T):
    depth, mask, idx, wu, wv = _run_geometry(
        plane_basis, plane_center, plane_wh, cam_R, cam_T)

    tex = plane_content.reshape(P, TEXN)
    to_chunks = lambda x: x.reshape(P, _NCHUNK, 1, _CHUNK)
    packed_in = jnp.concatenate(
        [to_chunks(idx.astype(jnp.float32)),
         to_chunks(wu), to_chunks(wv), to_chunks(mask)], axis=2)
    packed_out = _sc_gather_fn()(tex, packed_in)
    alpha, red, grn, blu = (packed_out[:, :, ch] for ch in range(4))

    to_img = lambda x: x.reshape(P, IMG_H, IMG_W)
    weight, sort_idx, rimg, gimg, bimg, depth_img = _run_composite(
        depth, to_img(alpha), to_img(red), to_img(grn), to_img(blu))

    color_img = jnp.stack([rimg, gimg, bimg], axis=-1)
    return (color_img, depth_img,
            weight.reshape(P, N), depth.reshape(P, N),
            mask.reshape(P, N).astype(bool), sort_idx.reshape(P, N))


# speedup vs baseline: 1.4142x; 1.4142x over previous
import dataclasses
import functools

import jax
import jax.numpy as jnp
from jax import lax
from jax.experimental import pallas as pl
from jax.experimental.pallas import tpu as pltpu
from jax.experimental.pallas import tpu_sc as plsc

P = 32
RES = 128
TEXN = 4 * RES * RES
IMG_H = IMG_W = 256
N = IMG_H * IMG_W
STEP = 2.0 / 255.0
ALMOST_ONE = 0.99999994


def _bitonic_pairs(n):
    pairs = []
    k = 2
    while k <= n:
        j = k // 2
        while j >= 1:
            for i in range(n):
                l = i ^ j
                if l > i:
                    pairs.append((i, l, (i & k) == 0))
            j //= 2
        k *= 2
    return pairs


_PAIRS = _bitonic_pairs(P)


_TILE_H = 8
_GRID_A = IMG_H // _TILE_H


def _b16(x):
    i = lax.bitcast_convert_type(x, jnp.int32)
    lsb = lax.bitwise_and(lax.shift_right_logical(i, 16), 1)
    r = lax.bitwise_and(i + (32767 + lsb), jnp.int32(-65536))
    return lax.bitcast_convert_type(r, jnp.float32)


def _two_sum(a, b):
    s = a + b
    bp = s - a
    ap = s - bp
    return s, (a - ap) + (b - bp)


def _dot3(p0, p1, p2):
    s1, e1 = _two_sum(p0, p1)
    s2, e2 = _two_sum(s1, p2)
    return s2 + (e1 + e2)


def _geom_body(par_ref, o_ref, rb_ref,
               depth_ref, mask_ref, idx_ref, wu_ref, wv_ref):
    gi = pl.program_id(0)
    o = [o_ref[j] for j in range(3)]
    yi = (gi * _TILE_H
          + lax.broadcasted_iota(jnp.int32, (_TILE_H, IMG_W), 0))
    xi = lax.broadcasted_iota(jnp.int32, (_TILE_H, IMG_W), 1)
    sx = xi.astype(jnp.float32) / 255.0
    sy = yi.astype(jnp.float32) / 255.0
    gx = sx - (1.0 - sx)
    gy = sy - (1.0 - sy)
    gxb = _b16(gx)
    gyb = _b16(gy)
    d = [_dot3(gxb * rb_ref[0, k], gyb * rb_ref[1, k],
               jnp.full((_TILE_H, IMG_W), rb_ref[2, k], jnp.float32))
         for k in range(3)]
    db = [_b16(x) for x in d]
    for p in range(P):
        nb = [par_ref[p, j] for j in range(3)]
        nump = par_ref[p, 3]
        B0 = [par_ref[p, 4 + j] for j in range(3)]
        B1 = [par_ref[p, 7 + j] for j in range(3)]
        c = [par_ref[p, 10 + j] for j in range(3)]
        hw0 = par_ref[p, 13]
        hw1 = par_ref[p, 14]
        den = _dot3(nb[0] * db[0], nb[1] * db[1], nb[2] * db[2])
        den = jnp.where(jnp.abs(den) < 1e-8, jnp.float32(1e-8), den)
        dep = nump / den
        wm = [_b16((o[j] + dep * d[j]) - c[j]) for j in range(3)]
        loc0 = _dot3(B0[0] * wm[0], B0[1] * wm[1], B0[2] * wm[2])
        loc1 = _dot3(B1[0] * wm[0], B1[1] * wm[1], B1[2] * wm[2])
        gxp = loc0 / hw0
        gyp = loc1 / hw1
        ip = (jnp.abs(gxp) <= 1.0) & (jnp.abs(gyp) <= 1.0)
        u = (gxp + 1.0) * 0.5 * (RES - 1)
        v = (gyp + 1.0) * 0.5 * (RES - 1)
        u0 = jnp.clip(jnp.floor(u), 0.0, RES - 1)
        v0 = jnp.clip(jnp.floor(v), 0.0, RES - 1)
        depth_ref[p] = dep
        mask_ref[p] = jnp.where(ip, jnp.float32(1.0), jnp.float32(0.0))
        idx_ref[p] = v0.astype(jnp.int32) * RES + u0.astype(jnp.int32)
        wu_ref[p] = jnp.clip(u - u0, 0.0, 1.0)
        wv_ref[p] = jnp.clip(v - v0, 0.0, 1.0)


def _run_geometry(plane_basis, plane_center, plane_wh, cam_R, cam_T):
    origin = -(cam_R.T @ cam_T)
    normals = plane_basis[:, 2, :]
    num = jnp.sum(normals * (plane_center - origin[None]), axis=-1)
    params = jnp.concatenate([
        _b16(normals),
        num[:, None],
        _b16(plane_basis[:, 0, :]),
        _b16(plane_basis[:, 1, :]),
        plane_center,
        plane_wh * 0.5,
    ], axis=1)
    smem = pl.BlockSpec(memory_space=pltpu.SMEM)
    obs = pl.BlockSpec((P, _TILE_H, IMG_W), lambda i: (0, i, 0))
    f32 = jnp.float32
    outs = jax.ShapeDtypeStruct((P, IMG_H, IMG_W), f32)
    return pl.pallas_call(
        _geom_body,
        grid=(_GRID_A,),
        in_specs=[smem] * 3,
        out_specs=[obs] * 5,
        out_shape=[outs, outs,
                   jax.ShapeDtypeStruct((P, IMG_H, IMG_W), jnp.int32),
                   outs, outs],
    )(params, origin, _b16(cam_R))


_NC = 2
_NS = 16
_CHUNK = 4096
_NCHUNK = N // _CHUNK


def _sc_gather_fn():
    mesh = plsc.VectorSubcoreMesh(core_axis_name="c", subcore_axis_name="s")
    f32 = jnp.float32
    out = jax.ShapeDtypeStruct((P, _NCHUNK, _CHUNK), f32)
    cp = pltpu.CompilerParams()
    if "needs_layout_passes" in pltpu.CompilerParams.__dataclass_fields__:
        cp = dataclasses.replace(cp, needs_layout_passes=False)

    @functools.partial(
        pl.kernel, mesh=mesh,
        out_type=[out, out, out, out],
        compiler_params=cp,
        scratch_types=[
            pltpu.VMEM((TEXN,), f32),
            pltpu.VMEM((_CHUNK,), jnp.int32),
            pltpu.VMEM((_CHUNK,), f32),
            pltpu.VMEM((_CHUNK,), f32),
            pltpu.VMEM((_CHUNK,), f32),
            pltpu.VMEM((_CHUNK,), f32),
            pltpu.VMEM((_CHUNK,), f32),
            pltpu.VMEM((_CHUNK,), f32),
            pltpu.VMEM((_CHUNK,), f32),
        ])
    def sc_gather(tex_hbm, idx_hbm, wu_hbm, wv_hbm, m_hbm,
                  a_hbm, r_hbm, g_hbm, b_hbm,
                  tex_v, idx_v, wu_v, wv_v, m_v, oa_v, or_v, og_v, ob_v):
        w = lax.axis_index("s") * _NC + lax.axis_index("c")
        pltpu.sync_copy(tex_hbm.at[w], tex_v)

        @pl.loop(0, _NCHUNK)
        def _chunk(ci):
            pltpu.sync_copy(idx_hbm.at[w, ci], idx_v)
            pltpu.sync_copy(wu_hbm.at[w, ci], wu_v)
            pltpu.sync_copy(wv_hbm.at[w, ci], wv_v)
            pltpu.sync_copy(m_hbm.at[w, ci], m_v)

            @plsc.parallel_loop(0, _CHUNK, step=16, unroll=4)
            def _grp(g):
                sl = pl.ds(g, 16)
                i00 = idx_v[sl]
                wu = wu_v[sl]
                wv = wv_v[sl]
                m = m_v[sl]
                u0 = lax.bitwise_and(i00, 127)
                du = jnp.where(u0 < 127, 1, 0).astype(jnp.int32)
                dv = jnp.where(i00 < (RES - 1) * RES, RES, 0).astype(jnp.int32)
                i10 = i00 + du
                i01 = i00 + dv
                i11 = i10 + dv
                w11 = wu * wv
                w10 = wu - w11
                w01 = wv - w11
                w00 = (1.0 - wu) - w01
                for ch, oref in enumerate((oa_v, or_v, og_v, ob_v)):
                    off = ch * (RES * RES)
                    c00 = plsc.load_gather(tex_v, [i00 + off])
                    c10 = plsc.load_gather(tex_v, [i10 + off])
                    c01 = plsc.load_gather(tex_v, [i01 + off])
                    c11 = plsc.load_gather(tex_v, [i11 + off])
                    val = c00 * w00 + c10 * w10 + c01 * w01 + c11 * w11
                    oref[sl] = m / (1.0 + jnp.exp(-val))

            pltpu.sync_copy(oa_v, a_hbm.at[w, ci])
            pltpu.sync_copy(or_v, r_hbm.at[w, ci])
            pltpu.sync_copy(og_v, g_hbm.at[w, ci])
            pltpu.sync_copy(ob_v, b_hbm.at[w, ci])

    return sc_gather


_TILE_CH = 8
_TILE_CW = 128


def _composite_body(d_ref, a_ref, r_ref, g_ref, b_ref,
                    w_ref, si_ref, ri_ref, gi_ref, bi_ref, di_ref):
    d = [d_ref[p] for p in range(P)]
    pay = [jnp.float32(p) + jnp.minimum(a_ref[p], jnp.float32(ALMOST_ONE))
           for p in range(P)]
    for i, l, asc in _PAIRS:
        c = (d[i] < d[l]) | ((d[i] == d[l]) & (pay[i] < pay[l]))
        if not asc:
            c = jnp.logical_not(c)
        d[i], d[l] = jnp.where(c, d[i], d[l]), jnp.where(c, d[l], d[i])
        pay[i], pay[l] = (jnp.where(c, pay[i], pay[l]),
                          jnp.where(c, pay[l], pay[i]))
    t = jnp.ones((_TILE_CH, _TILE_CW), jnp.float32)
    dimg = jnp.zeros((_TILE_CH, _TILE_CW), jnp.float32)
    key2 = []
    for s in range(P):
        idxf = jnp.floor(pay[s])
        alpha = pay[s] - idxf
        wgt = alpha * t
        t = t * (1.0 - alpha)
        w_ref[s] = wgt
        si_ref[s] = pay[s].astype(jnp.int32)
        dimg = dimg + d[s] * wgt
        key2.append(idxf + jnp.minimum(wgt, jnp.float32(ALMOST_ONE)))
    for i, l, asc in _PAIRS:
        lo = jnp.minimum(key2[i], key2[l])
        hi = jnp.maximum(key2[i], key2[l])
        key2[i], key2[l] = (lo, hi) if asc else (hi, lo)
    rimg = jnp.zeros((_TILE_CH, _TILE_CW), jnp.float32)
    gimg = jnp.zeros((_TILE_CH, _TILE_CW), jnp.float32)
    bimg = jnp.zeros((_TILE_CH, _TILE_CW), jnp.float32)
    for p in range(P):
        wo = key2[p] - jnp.float32(p)
        rimg = rimg + r_ref[p] * wo
        gimg = gimg + g_ref[p] * wo
        bimg = bimg + b_ref[p] * wo
    ri_ref[...] = rimg
    gi_ref[...] = gimg
    bi_ref[...] = bimg
    di_ref[...] = dimg


def _run_composite(depth, alpha, red, grn, blu):
    ibs = pl.BlockSpec((P, _TILE_CH, _TILE_CW), lambda i, j: (0, i, j))
    img = pl.BlockSpec((_TILE_CH, _TILE_CW), lambda i, j: (i, j))
    f32 = jnp.float32
    return pl.pallas_call(
        _composite_body,
        grid=(IMG_H // _TILE_CH, IMG_W // _TILE_CW),
        in_specs=[ibs] * 5,
        out_specs=[ibs, ibs, img, img, img, img],
        out_shape=[
            jax.ShapeDtypeStruct((P, IMG_H, IMG_W), f32),
            jax.ShapeDtypeStruct((P, IMG_H, IMG_W), jnp.int32),
            jax.ShapeDtypeStruct((IMG_H, IMG_W), f32),
            jax.ShapeDtypeStruct((IMG_H, IMG_W), f32),
            jax.ShapeDtypeStruct((IMG_H, IMG_W), f32),
            jax.ShapeDtypeStruct((IMG_H, IMG_W), f32),
        ],
    )(depth, alpha, red, grn, blu)


def kernel(plane_content, plane_basis, plane_center, plane_wh, cam_R, cam_T):
    depth, mask, idx, wu, wv = _run_geometry(
        plane_basis, plane_center, plane_wh, cam_R, cam_T)

    tex = plane_content.reshape(P, TEXN)
    to_chunks = lambda x: x.reshape(P, _NCHUNK, _CHUNK)
    alpha, red, grn, blu = _sc_gather_fn()(
        tex, to_chunks(idx), to_chunks(wu), to_chunks(wv), to_chunks(mask))

    to_img = lambda x: x.reshape(P, IMG_H, IMG_W)
    weight, sort_idx, rimg, gimg, bimg, depth_img = _run_composite(
        depth, to_img(alpha), to_img(red), to_img(grn), to_img(blu))

    color_img = jnp.stack([rimg, gimg, bimg], axis=-1)
    return (color_img, depth_img,
            weight.reshape(P, N), depth.reshape(P, N),
            mask.reshape(P, N).astype(bool), sort_idx.reshape(P, N))


# speedup vs baseline: 1.4631x; 1.0346x over previous
import dataclasses
import functools

import jax
import jax.numpy as jnp
from jax import lax
from jax.experimental import pallas as pl
from jax.experimental.pallas import tpu as pltpu
from jax.experimental.pallas import tpu_sc as plsc

P = 32
RES = 128
TEXN = 4 * RES * RES
IMG_H = IMG_W = 256
N = IMG_H * IMG_W
STEP = 2.0 / 255.0
ALMOST_ONE = 0.99999994


def _bitonic_pairs(n):
    pairs = []
    k = 2
    while k <= n:
        j = k // 2
        while j >= 1:
            for i in range(n):
                l = i ^ j
                if l > i:
                    pairs.append((i, l, (i & k) == 0))
            j //= 2
        k *= 2
    return pairs


_PAIRS = _bitonic_pairs(P)


_TILE_H = 8
_GRID_A = IMG_H // _TILE_H


def _b16(x):
    i = lax.bitcast_convert_type(x, jnp.int32)
    lsb = lax.bitwise_and(lax.shift_right_logical(i, 16), 1)
    r = lax.bitwise_and(i + (32767 + lsb), jnp.int32(-65536))
    return lax.bitcast_convert_type(r, jnp.float32)


def _two_sum(a, b):
    s = a + b
    bp = s - a
    ap = s - bp
    return s, (a - ap) + (b - bp)


def _dot3(p0, p1, p2):
    s1, e1 = _two_sum(p0, p1)
    s2, e2 = _two_sum(s1, p2)
    return s2 + (e1 + e2)


def _geom_body(par_ref, o_ref, rb_ref,
               depth_ref, mask_ref, idx_ref, wu_ref, wv_ref):
    gi = pl.program_id(0)
    o = [o_ref[j] for j in range(3)]
    yi = (gi * _TILE_H
          + lax.broadcasted_iota(jnp.int32, (_TILE_H, IMG_W), 0))
    xi = lax.broadcasted_iota(jnp.int32, (_TILE_H, IMG_W), 1)
    sx = xi.astype(jnp.float32) / 255.0
    sy = yi.astype(jnp.float32) / 255.0
    gx = sx - (1.0 - sx)
    gy = sy - (1.0 - sy)
    gxb = _b16(gx)
    gyb = _b16(gy)
    d = [_dot3(gxb * rb_ref[0, k], gyb * rb_ref[1, k],
               jnp.full((_TILE_H, IMG_W), rb_ref[2, k], jnp.float32))
         for k in range(3)]
    db = [_b16(x) for x in d]
    for p in range(P):
        nb = [par_ref[p, j] for j in range(3)]
        nump = par_ref[p, 3]
        B0 = [par_ref[p, 4 + j] for j in range(3)]
        B1 = [par_ref[p, 7 + j] for j in range(3)]
        c = [par_ref[p, 10 + j] for j in range(3)]
        hw0 = par_ref[p, 13]
        hw1 = par_ref[p, 14]
        den = _dot3(nb[0] * db[0], nb[1] * db[1], nb[2] * db[2])
        den = jnp.where(jnp.abs(den) < 1e-8, jnp.float32(1e-8), den)
        dep = nump / den
        wm = [_b16((o[j] + dep * d[j]) - c[j]) for j in range(3)]
        loc0 = _dot3(B0[0] * wm[0], B0[1] * wm[1], B0[2] * wm[2])
        loc1 = _dot3(B1[0] * wm[0], B1[1] * wm[1], B1[2] * wm[2])
        gxp = loc0 / hw0
        gyp = loc1 / hw1
        ip = (jnp.abs(gxp) <= 1.0) & (jnp.abs(gyp) <= 1.0)
        u = (gxp + 1.0) * 0.5 * (RES - 1)
        v = (gyp + 1.0) * 0.5 * (RES - 1)
        u0 = jnp.clip(jnp.floor(u), 0.0, RES - 1)
        v0 = jnp.clip(jnp.floor(v), 0.0, RES - 1)
        depth_ref[p] = dep
        mask_ref[p] = jnp.where(ip, jnp.float32(1.0), jnp.float32(0.0))
        idx_ref[p] = v0.astype(jnp.int32) * RES + u0.astype(jnp.int32)
        wu_ref[p] = jnp.clip(u - u0, 0.0, 1.0)
        wv_ref[p] = jnp.clip(v - v0, 0.0, 1.0)


def _run_geometry(plane_basis, plane_center, plane_wh, cam_R, cam_T):
    origin = -(cam_R.T @ cam_T)
    normals = plane_basis[:, 2, :]
    num = jnp.sum(normals * (plane_center - origin[None]), axis=-1)
    params = jnp.concatenate([
        _b16(normals),
        num[:, None],
        _b16(plane_basis[:, 0, :]),
        _b16(plane_basis[:, 1, :]),
        plane_center,
        plane_wh * 0.5,
    ], axis=1)
    smem = pl.BlockSpec(memory_space=pltpu.SMEM)
    obs = pl.BlockSpec((P, _TILE_H, IMG_W), lambda i: (0, i, 0))
    f32 = jnp.float32
    outs = jax.ShapeDtypeStruct((P, IMG_H, IMG_W), f32)
    return pl.pallas_call(
        _geom_body,
        grid=(_GRID_A,),
        in_specs=[smem] * 3,
        out_specs=[obs] * 5,
        out_shape=[outs, outs,
                   jax.ShapeDtypeStruct((P, IMG_H, IMG_W), jnp.int32),
                   outs, outs],
    )(params, origin, _b16(cam_R))


_NC = 2
_NS = 16
_CHUNK = 4096
_NCHUNK = N // _CHUNK


def _sc_gather_fn():
    mesh = plsc.VectorSubcoreMesh(core_axis_name="c", subcore_axis_name="s")
    f32 = jnp.float32
    out = jax.ShapeDtypeStruct((P, _NCHUNK, _CHUNK), f32)
    cp = pltpu.CompilerParams()
    if "needs_layout_passes" in pltpu.CompilerParams.__dataclass_fields__:
        cp = dataclasses.replace(cp, needs_layout_passes=False)

    @functools.partial(
        pl.kernel, mesh=mesh,
        out_type=[out, out, out, out],
        compiler_params=cp,
        scratch_types=[
            pltpu.VMEM((TEXN,), f32),
            pltpu.VMEM((_CHUNK,), jnp.int32),
            pltpu.VMEM((_CHUNK,), f32),
            pltpu.VMEM((_CHUNK,), f32),
            pltpu.VMEM((_CHUNK,), f32),
            pltpu.VMEM((_CHUNK,), f32),
            pltpu.VMEM((_CHUNK,), f32),
            pltpu.VMEM((_CHUNK,), f32),
            pltpu.VMEM((_CHUNK,), f32),
            pltpu.SemaphoreType.DMA,
            pltpu.SemaphoreType.DMA,
        ])
    def sc_gather(tex_hbm, idx_hbm, wu_hbm, wv_hbm, m_hbm,
                  a_hbm, r_hbm, g_hbm, b_hbm,
                  tex_v, idx_v, wu_v, wv_v, m_v, oa_v, or_v, og_v, ob_v,
                  sem_i, sem_o):
        w = lax.axis_index("s") * _NC + lax.axis_index("c")
        pltpu.sync_copy(tex_hbm.at[w], tex_v)

        @pl.loop(0, _NCHUNK)
        def _chunk(ci):
            hs = [pltpu.async_copy(idx_hbm.at[w, ci], idx_v, sem_i),
                  pltpu.async_copy(wu_hbm.at[w, ci], wu_v, sem_i),
                  pltpu.async_copy(wv_hbm.at[w, ci], wv_v, sem_i),
                  pltpu.async_copy(m_hbm.at[w, ci], m_v, sem_i)]
            for h in hs:
                h.wait()

            @plsc.parallel_loop(0, _CHUNK, step=16, unroll=4)
            def _grp(g):
                sl = pl.ds(g, 16)
                i00 = idx_v[sl]
                wu = wu_v[sl]
                wv = wv_v[sl]
                m = m_v[sl]
                u0 = lax.bitwise_and(i00, 127)
                du = jnp.where(u0 < 127, 1, 0).astype(jnp.int32)
                dv = jnp.where(i00 < (RES - 1) * RES, RES, 0).astype(jnp.int32)
                i10 = i00 + du
                i01 = i00 + dv
                i11 = i10 + dv
                w11 = wu * wv
                w10 = wu - w11
                w01 = wv - w11
                w00 = (1.0 - wu) - w01
                for ch, oref in enumerate((oa_v, or_v, og_v, ob_v)):
                    off = ch * (RES * RES)
                    c00 = plsc.load_gather(tex_v, [i00 + off])
                    c10 = plsc.load_gather(tex_v, [i10 + off])
                    c01 = plsc.load_gather(tex_v, [i01 + off])
                    c11 = plsc.load_gather(tex_v, [i11 + off])
                    val = c00 * w00 + c10 * w10 + c01 * w01 + c11 * w11
                    oref[sl] = m / (1.0 + jnp.exp(-val))

            os_ = [pltpu.async_copy(oa_v, a_hbm.at[w, ci], sem_o),
                   pltpu.async_copy(or_v, r_hbm.at[w, ci], sem_o),
                   pltpu.async_copy(og_v, g_hbm.at[w, ci], sem_o),
                   pltpu.async_copy(ob_v, b_hbm.at[w, ci], sem_o)]
            for h in os_:
                h.wait()

    return sc_gather


_TILE_CH = 8
_TILE_CW = 128


def _composite_body(d_ref, a_ref, r_ref, g_ref, b_ref,
                    w_ref, si_ref, ri_ref, gi_ref, bi_ref, di_ref):
    d = [d_ref[p] for p in range(P)]
    pay = [jnp.float32(p) + jnp.minimum(a_ref[p], jnp.float32(ALMOST_ONE))
           for p in range(P)]
    for i, l, asc in _PAIRS:
        c = (d[i] < d[l]) | ((d[i] == d[l]) & (pay[i] < pay[l]))
        if not asc:
            c = jnp.logical_not(c)
        d[i], d[l] = jnp.where(c, d[i], d[l]), jnp.where(c, d[l], d[i])
        pay[i], pay[l] = (jnp.where(c, pay[i], pay[l]),
                          jnp.where(c, pay[l], pay[i]))
    t = jnp.ones((_TILE_CH, _TILE_CW), jnp.float32)
    dimg = jnp.zeros((_TILE_CH, _TILE_CW), jnp.float32)
    key2 = []
    for s in range(P):
        idxf = jnp.floor(pay[s])
        alpha = pay[s] - idxf
        wgt = alpha * t
        t = t * (1.0 - alpha)
        w_ref[s] = wgt
        si_ref[s] = pay[s].astype(jnp.int32)
        dimg = dimg + d[s] * wgt
        key2.append(idxf + jnp.minimum(wgt, jnp.float32(ALMOST_ONE)))
    for i, l, asc in _PAIRS:
        lo = jnp.minimum(key2[i], key2[l])
        hi = jnp.maximum(key2[i], key2[l])
        key2[i], key2[l] = (lo, hi) if asc else (hi, lo)
    rimg = jnp.zeros((_TILE_CH, _TILE_CW), jnp.float32)
    gimg = jnp.zeros((_TILE_CH, _TILE_CW), jnp.float32)
    bimg = jnp.zeros((_TILE_CH, _TILE_CW), jnp.float32)
    for p in range(P):
        wo = key2[p] - jnp.float32(p)
        rimg = rimg + r_ref[p] * wo
        gimg = gimg + g_ref[p] * wo
        bimg = bimg + b_ref[p] * wo
    ri_ref[...] = rimg
    gi_ref[...] = gimg
    bi_ref[...] = bimg
    di_ref[...] = dimg


def _run_composite(depth, alpha, red, grn, blu):
    ibs = pl.BlockSpec((P, _TILE_CH, _TILE_CW), lambda i, j: (0, i, j))
    img = pl.BlockSpec((_TILE_CH, _TILE_CW), lambda i, j: (i, j))
    f32 = jnp.float32
    return pl.pallas_call(
        _composite_body,
        grid=(IMG_H // _TILE_CH, IMG_W // _TILE_CW),
        in_specs=[ibs] * 5,
        out_specs=[ibs, ibs, img, img, img, img],
        out_shape=[
            jax.ShapeDtypeStruct((P, IMG_H, IMG_W), f32),
            jax.ShapeDtypeStruct((P, IMG_H, IMG_W), jnp.int32),
            jax.ShapeDtypeStruct((IMG_H, IMG_W), f32),
            jax.ShapeDtypeStruct((IMG_H, IMG_W), f32),
            jax.ShapeDtypeStruct((IMG_H, IMG_W), f32),
            jax.ShapeDtypeStruct((IMG_H, IMG_W), f32),
        ],
    )(depth, alpha, red, grn, blu)


def kernel(plane_content, plane_basis, plane_center, plane_wh, cam_R, cam_T):
    depth, mask, idx, wu, wv = _run_geometry(
        plane_basis, plane_center, plane_wh, cam_R, cam_T)

    tex = plane_content.reshape(P, TEXN)
    to_chunks = lambda x: x.reshape(P, _NCHUNK, _CHUNK)
    alpha, red, grn, blu = _sc_gather_fn()(
        tex, to_chunks(idx), to_chunks(wu), to_chunks(wv), to_chunks(mask))

    to_img = lambda x: x.reshape(P, IMG_H, IMG_W)
    weight, sort_idx, rimg, gimg, bimg, depth_img = _run_composite(
        depth, to_img(alpha), to_img(red), to_img(grn), to_img(blu))

    color_img = jnp.stack([rimg, gimg, bimg], axis=-1)
    return (color_img, depth_img,
            weight.reshape(P, N), depth.reshape(P, N),
            mask.reshape(P, N).astype(bool), sort_idx.reshape(P, N))
